# baseline (speedup 1.0000x reference)
import functools

import jax
import jax.numpy as jnp
from jax import lax
from jax.experimental import pallas as pl
from jax.experimental.pallas import tpu as pltpu
from jax.experimental.pallas import tpu_sc as plsc

N = 10000
E = 160000
D_IN = 128
D_H1 = 400
D_H2 = 800

C = 80
KC1 = D_H1 // C
KC2 = D_H2 // C

EB = 128
E_PAD = 163840
NB16 = E_PAD // 16 // EB
NB32 = E_PAD // 32 // EB
TRASH = N

ROWS_PER_TILE = N // 16

DEG_TILE = 626
DEG_ROWS = 16 * DEG_TILE

NBUF = 2
NQ = 4
HQ = 2
QB = NB16 // NQ
QGRP = QB // NBUF

_MESH = plsc.VectorSubcoreMesh(core_axis_name="c", subcore_axis_name="s")
_SC_PARAMS = pltpu.CompilerParams(use_tc_tiling_on_sc=False)


@functools.partial(
    pl.kernel,
    mesh=_MESH,
    out_type=jax.ShapeDtypeStruct((2, DEG_ROWS, 16), jnp.float32),
    scratch_types=[
        pltpu.VMEM_SHARED((DEG_ROWS, 16), jnp.float32),
        pltpu.VMEM((NB32, EB), jnp.int32),
        pltpu.VMEM((EB, 16), jnp.float32),
        pltpu.VMEM((DEG_TILE, 16), jnp.float32),
    ],
    compiler_params=_SC_PARAMS,
)
def _deg_kernel(dst_hbm, out_hbm, deg_sp, idx_v, ones_v, buf_v):
    c = lax.axis_index("c")
    s = lax.axis_index("s")
    w = c * 16 + s

    def fill_ones(i, _):
        ones_v[i, :] = jnp.full((16,), 1.0, jnp.float32)
        return 0

    lax.fori_loop(0, EB, fill_ones, 0)

    def fill_zero(i, _):
        buf_v[i, :] = jnp.zeros((16,), jnp.float32)
        return 0

    lax.fori_loop(0, DEG_TILE, fill_zero, 0)

    r0 = s * DEG_TILE
    pltpu.sync_copy(buf_v, deg_sp.at[pl.ds(r0, DEG_TILE)])
    pltpu.sync_copy(dst_hbm.at[w], idx_v)
    plsc.subcore_barrier()

    def edge(b, _):
        pltpu.sync_copy(ones_v, deg_sp.at[idx_v.at[b]], add=True)
        return 0

    lax.fori_loop(0, NB32, edge, 0)
    plsc.subcore_barrier()

    pltpu.sync_copy(deg_sp.at[pl.ds(r0, DEG_TILE)], buf_v)
    pltpu.sync_copy(buf_v, out_hbm.at[c].at[pl.ds(r0, DEG_TILE)])


def _make_scatter(kc_total):
    split_last = kc_total % 2 == 1
    kcc = kc_total // 2
    n_out = kc_total + 1 if split_last else kc_total

    @functools.partial(
        pl.kernel,
        mesh=_MESH,
        out_type=jax.ShapeDtypeStruct((n_out, N, C), jnp.float32),
        scratch_types=[
            pltpu.VMEM_SHARED((N, C), jnp.float32),
            pltpu.VMEM_SHARED((N + 1, C), jnp.float32),
            pltpu.VMEM((QB, EB), jnp.int32),
            pltpu.VMEM((QB, EB), jnp.int32),
            pltpu.VMEM((NBUF, EB, C), jnp.float32),
            pltpu.SemaphoreType.DMA((NBUF,)),
            pltpu.SemaphoreType.DMA((NBUF,)),
        ],
        compiler_params=_SC_PARAMS,
    )
    def scatter(g_hbm, src_hbm, dst_hbm, out_hbm,
                tab_sp, acc_sp, src_v, dst_v, row_v, sem_g, sem_s):
        c = lax.axis_index("c")
        s = lax.axis_index("s")
        r0 = s * ROWS_PER_TILE
        rows = pl.ds(r0, ROWS_PER_TILE)

        def run_chunk(k, out_slot, q_base, nq_local, hq_local):
            pltpu.sync_copy(g_hbm.at[k].at[rows], tab_sp.at[rows])
            pltpu.sync_copy(g_hbm.at[k].at[rows], acc_sp.at[rows])
            plsc.subcore_barrier()

            def g_start(i, b, from_hbm):
                src = (g_hbm.at[k] if from_hbm else tab_sp)
                pltpu.async_copy(src.at[src_v.at[b]], row_v.at[i],
                                 sem_g.at[i])

            def g_wait(i, b, from_hbm):
                src = (g_hbm.at[k] if from_hbm else tab_sp)
                pltpu.make_async_copy(src.at[src_v.at[b]], row_v.at[i],
                                      sem_g.at[i]).wait()

            def s_start(i, b):
                pltpu.async_copy(row_v.at[i], acc_sp.at[dst_v.at[b]],
                                 sem_s.at[i], add=True)

            def s_wait(i, b):
                pltpu.make_async_copy(row_v.at[i], acc_sp.at[dst_v.at[b]],
                                      sem_s.at[i]).wait()

            for q in range(nq_local):
                q_off = (q_base + q) * QB
                pltpu.sync_copy(src_hbm.at[s].at[pl.ds(q_off, QB)], src_v)
                pltpu.sync_copy(dst_hbm.at[s].at[pl.ds(q_off, QB)], dst_v)
                for i in range(NBUF):
                    g_start(i, i, i % 2 == 0)

                def edge_grp(g, _):
                    for i in range(NBUF):
                        g_wait(i, g * NBUF + i, i % 2 == 0)
                        s_start(i, g * NBUF + i)
                    for i in range(NBUF):
                        s_wait(i, g * NBUF + i)
                        g_start(i, (g + 1) * NBUF + i, i % 2 == 0)
                    return 0

                lax.fori_loop(0, QGRP - 1, edge_grp, 0)
                for i in range(NBUF):
                    b = (QGRP - 1) * NBUF + i
                    g_wait(i, b, i % 2 == 0)
                    s_start(i, b)
                for i in range(NBUF):
                    s_wait(i, (QGRP - 1) * NBUF + i)
            plsc.subcore_barrier()

            pltpu.sync_copy(acc_sp.at[rows], out_hbm.at[out_slot].at[rows])
            plsc.subcore_barrier()

        def chunk(kc, _):
            k = kc * 2 + c
            run_chunk(k, k, jnp.int32(0), NQ, HQ)
            return 0

        lax.fori_loop(0, kcc, chunk, 0)
        if split_last:
            run_chunk(kc_total - 1, kc_total - 1 + c, c * (NQ // 2),
                      NQ // 2, HQ // 2)

    return scatter


_scatter_l1 = _make_scatter(KC1)
_scatter_l2 = _make_scatter(KC2)


def _dinv_of(deg_ref):
    return lax.rsqrt(deg_ref[0, :, :1] + deg_ref[1, :, :1] + 1.0)


def _mm1_body(x_ref, w_ref, deg_ref, o_ref):
    dinv = _dinv_of(deg_ref)
    g = jnp.dot(x_ref[...], w_ref[...],
                preferred_element_type=jnp.float32) * dinv
    for k in range(KC1):
        o_ref[k] = g[:, k * C:(k + 1) * C]


def _mm1(x, w1, deg2):
    return pl.pallas_call(
        _mm1_body,
        grid=(25,),
        in_specs=[
            pl.BlockSpec((400, D_IN), lambda i: (i, 0)),
            pl.BlockSpec((D_IN, D_H1), lambda i: (0, 0)),
            pl.BlockSpec((2, 400, 16), lambda i: (0, i, 0)),
        ],
        out_specs=pl.BlockSpec((KC1, 400, C), lambda i: (0, i, 0)),
        out_shape=jax.ShapeDtypeStruct((KC1, N, C), jnp.float32),
    )(x, w1, deg2)


def _mm2_body(a_ref, deg_ref, g4_ref, b1_ref, w_ref, o_ref):
    dinv = _dinv_of(deg_ref)
    parts = [a_ref[k] for k in range(KC1 - 1)]
    parts.append(a_ref[KC1 - 1] + a_ref[KC1] - g4_ref[0])
    h = jnp.concatenate(parts, axis=1)
    h = jnp.maximum(h * dinv + b1_ref[...], 0.0)
    g = jnp.dot(h, w_ref[...], preferred_element_type=jnp.float32) * dinv
    for k in range(KC2):
        o_ref[k] = g[:, k * C:(k + 1) * C]


def _mm2(acc1, deg2, g1, b1, w2):
    return pl.pallas_call(
        _mm2_body,
        grid=(25,),
        in_specs=[
            pl.BlockSpec((KC1 + 1, 400, C), lambda i: (0, i, 0)),
            pl.BlockSpec((2, 400, 16), lambda i: (0, i, 0)),
            pl.BlockSpec((1, 400, C), lambda i: (KC1 - 1, i, 0)),
            pl.BlockSpec((1, D_H1), lambda i: (0, 0)),
            pl.BlockSpec((D_H1, D_H2), lambda i: (0, 0)),
        ],
        out_specs=pl.BlockSpec((KC2, 400, C), lambda i: (0, i, 0)),
        out_shape=jax.ShapeDtypeStruct((KC2, N, C), jnp.float32),
    )(acc1, deg2, g1, b1, w2)


def _final_body(a_ref, deg_ref, b2_ref, o_ref):
    dinv = _dinv_of(deg_ref)
    acc = jnp.concatenate([a_ref[k] for k in range(KC2)], axis=1)
    o_ref[...] = acc * dinv + b2_ref[...]


def _final(acc2, deg2, b2):
    return pl.pallas_call(
        _final_body,
        grid=(25,),
        in_specs=[
            pl.BlockSpec((KC2, 400, C), lambda i: (0, i, 0)),
            pl.BlockSpec((2, 400, 16), lambda i: (0, i, 0)),
            pl.BlockSpec((1, D_H2), lambda i: (0, 0)),
        ],
        out_specs=pl.BlockSpec((400, D_H2), lambda i: (i, 0)),
        out_shape=jax.ShapeDtypeStruct((N, D_H2), jnp.float32),
    )(acc2, deg2, b2)


def kernel(x, edge_index, W1, b1, W2, b2):
    src = edge_index[0].astype(jnp.int32)
    dst = edge_index[1].astype(jnp.int32)
    npad = E_PAD - E
    src_p = jnp.concatenate([src, jnp.zeros((npad,), jnp.int32)])
    dst_p = jnp.concatenate([dst, jnp.full((npad,), TRASH, jnp.int32)])
    src16 = src_p.reshape(16, NB16, EB)
    dst16 = dst_p.reshape(16, NB16, EB)
    dst32 = dst_p.reshape(32, NB32, EB)

    deg2 = _deg_kernel(dst32)[:, :N, :]
    g1 = _mm1(x, W1, deg2)
    acc1 = _scatter_l1(g1, src16, dst16)
    g2 = _mm2(acc1, deg2, g1, b1.reshape(1, D_H1), W2)
    acc2 = _scatter_l2(g2, src16, dst16)
    return _final(acc2, deg2, b2.reshape(1, D_H2))


# speedup vs baseline: 1.1264x; 1.1264x over previous
import functools

import jax
import jax.numpy as jnp
from jax import lax
from jax.experimental import pallas as pl
from jax.experimental.pallas import tpu as pltpu
from jax.experimental.pallas import tpu_sc as plsc

N = 10000
E = 160000
D_IN = 128
D_H1 = 400
D_H2 = 800

C = 80
KC1 = D_H1 // C
KC2 = D_H2 // C

EB = 64
E_PAD = 163840
NB16 = E_PAD // 16 // EB
NB32 = E_PAD // 32 // EB
TRASH = N

ROWS_PER_TILE = N // 16

DEG_TILE = 626
DEG_ROWS = 16 * DEG_TILE

NBUF = 4
NQ = 4
HQ = 2
QB = NB16 // NQ
QGRP = QB // NBUF

_MESH = plsc.VectorSubcoreMesh(core_axis_name="c", subcore_axis_name="s")
_SC_PARAMS = pltpu.CompilerParams(use_tc_tiling_on_sc=False)


@functools.partial(
    pl.kernel,
    mesh=_MESH,
    out_type=jax.ShapeDtypeStruct((2, DEG_ROWS, 16), jnp.float32),
    scratch_types=[
        pltpu.VMEM_SHARED((DEG_ROWS, 16), jnp.float32),
        pltpu.VMEM((NB32, EB), jnp.int32),
        pltpu.VMEM((EB, 16), jnp.float32),
        pltpu.VMEM((DEG_TILE, 16), jnp.float32),
    ],
    compiler_params=_SC_PARAMS,
)
def _deg_kernel(dst_hbm, out_hbm, deg_sp, idx_v, ones_v, buf_v):
    c = lax.axis_index("c")
    s = lax.axis_index("s")
    w = c * 16 + s

    def fill_ones(i, _):
        ones_v[i, :] = jnp.full((16,), 1.0, jnp.float32)
        return 0

    lax.fori_loop(0, EB, fill_ones, 0)

    def fill_zero(i, _):
        buf_v[i, :] = jnp.zeros((16,), jnp.float32)
        return 0

    lax.fori_loop(0, DEG_TILE, fill_zero, 0)

    r0 = s * DEG_TILE
    pltpu.sync_copy(buf_v, deg_sp.at[pl.ds(r0, DEG_TILE)])
    pltpu.sync_copy(dst_hbm.at[w], idx_v)
    plsc.subcore_barrier()

    def edge(b, _):
        pltpu.sync_copy(ones_v, deg_sp.at[idx_v.at[b]], add=True)
        return 0

    lax.fori_loop(0, NB32, edge, 0)
    plsc.subcore_barrier()

    pltpu.sync_copy(deg_sp.at[pl.ds(r0, DEG_TILE)], buf_v)
    pltpu.sync_copy(buf_v, out_hbm.at[c].at[pl.ds(r0, DEG_TILE)])


def _make_scatter(kc_total):
    split_last = kc_total % 2 == 1
    kcc = kc_total // 2
    n_out = kc_total + 1 if split_last else kc_total

    @functools.partial(
        pl.kernel,
        mesh=_MESH,
        out_type=jax.ShapeDtypeStruct((n_out, N, C), jnp.float32),
        scratch_types=[
            pltpu.VMEM_SHARED((N, C), jnp.float32),
            pltpu.VMEM_SHARED((N + 1, C), jnp.float32),
            pltpu.VMEM((QB, EB), jnp.int32),
            pltpu.VMEM((QB, EB), jnp.int32),
            pltpu.VMEM((NBUF, EB, C), jnp.float32),
            pltpu.SemaphoreType.DMA((NBUF,)),
            pltpu.SemaphoreType.DMA((NBUF,)),
        ],
        compiler_params=_SC_PARAMS,
    )
    def scatter(g_hbm, src_hbm, dst_hbm, out_hbm,
                tab_sp, acc_sp, src_v, dst_v, row_v, sem_g, sem_s):
        c = lax.axis_index("c")
        s = lax.axis_index("s")
        r0 = s * ROWS_PER_TILE
        rows = pl.ds(r0, ROWS_PER_TILE)

        def run_chunk(k, out_slot, q_base, nq_local, hq_local):
            pltpu.sync_copy(g_hbm.at[k].at[rows], tab_sp.at[rows])
            pltpu.sync_copy(g_hbm.at[k].at[rows], acc_sp.at[rows])
            plsc.subcore_barrier()

            def g_start(i, b, from_hbm):
                src = (g_hbm.at[k] if from_hbm else tab_sp)
                pltpu.async_copy(src.at[src_v.at[b]], row_v.at[i],
                                 sem_g.at[i])

            def g_wait(i, b, from_hbm):
                src = (g_hbm.at[k] if from_hbm else tab_sp)
                pltpu.make_async_copy(src.at[src_v.at[b]], row_v.at[i],
                                      sem_g.at[i]).wait()

            def s_start(i, b):
                pltpu.async_copy(row_v.at[i], acc_sp.at[dst_v.at[b]],
                                 sem_s.at[i], add=True)

            def s_wait(i, b):
                pltpu.make_async_copy(row_v.at[i], acc_sp.at[dst_v.at[b]],
                                      sem_s.at[i]).wait()

            for q in range(nq_local):
                q_off = (q_base + q) * QB
                pltpu.sync_copy(src_hbm.at[s].at[pl.ds(q_off, QB)], src_v)
                pltpu.sync_copy(dst_hbm.at[s].at[pl.ds(q_off, QB)], dst_v)
                for i in range(NBUF):
                    g_start(i, i, i < NBUF // 2)

                def edge_grp(g, _):
                    for i in range(NBUF):
                        g_wait(i, g * NBUF + i, i < NBUF // 2)
                        s_start(i, g * NBUF + i)
                    for i in range(NBUF):
                        s_wait(i, g * NBUF + i)
                        g_start(i, (g + 1) * NBUF + i, i < NBUF // 2)
                    return 0

                lax.fori_loop(0, QGRP - 1, edge_grp, 0)
                for i in range(NBUF):
                    b = (QGRP - 1) * NBUF + i
                    g_wait(i, b, i < NBUF // 2)
                    s_start(i, b)
                for i in range(NBUF):
                    s_wait(i, (QGRP - 1) * NBUF + i)
            plsc.subcore_barrier()

            pltpu.sync_copy(acc_sp.at[rows], out_hbm.at[out_slot].at[rows])
            plsc.subcore_barrier()

        def chunk(kc, _):
            k = kc * 2 + c
            run_chunk(k, k, jnp.int32(0), NQ, HQ)
            return 0

        lax.fori_loop(0, kcc, chunk, 0)
        if split_last:
            run_chunk(kc_total - 1, kc_total - 1 + c, c * (NQ // 2),
                      NQ // 2, HQ // 2)

    return scatter


_scatter_l1 = _make_scatter(KC1)
_scatter_l2 = _make_scatter(KC2)


def _dinv_of(deg_ref):
    return lax.rsqrt(deg_ref[0, :, :1] + deg_ref[1, :, :1] + 1.0)


def _mm1_body(x_ref, w_ref, deg_ref, o_ref):
    dinv = _dinv_of(deg_ref)
    g = jnp.dot(x_ref[...], w_ref[...],
                preferred_element_type=jnp.float32) * dinv
    for k in range(KC1):
        o_ref[k] = g[:, k * C:(k + 1) * C]


def _mm1(x, w1, deg2):
    return pl.pallas_call(
        _mm1_body,
        grid=(25,),
        in_specs=[
            pl.BlockSpec((400, D_IN), lambda i: (i, 0)),
            pl.BlockSpec((D_IN, D_H1), lambda i: (0, 0)),
            pl.BlockSpec((2, 400, 16), lambda i: (0, i, 0)),
        ],
        out_specs=pl.BlockSpec((KC1, 400, C), lambda i: (0, i, 0)),
        out_shape=jax.ShapeDtypeStruct((KC1, N, C), jnp.float32),
    )(x, w1, deg2)


def _mm2_body(a_ref, deg_ref, g4_ref, b1_ref, w_ref, o_ref):
    dinv = _dinv_of(deg_ref)
    parts = [a_ref[k] for k in range(KC1 - 1)]
    parts.append(a_ref[KC1 - 1] + a_ref[KC1] - g4_ref[0])
    h = jnp.concatenate(parts, axis=1)
    h = jnp.maximum(h * dinv + b1_ref[...], 0.0)
    g = jnp.dot(h, w_ref[...], preferred_element_type=jnp.float32) * dinv
    for k in range(KC2):
        o_ref[k] = g[:, k * C:(k + 1) * C]


def _mm2(acc1, deg2, g1, b1, w2):
    return pl.pallas_call(
        _mm2_body,
        grid=(25,),
        in_specs=[
            pl.BlockSpec((KC1 + 1, 400, C), lambda i: (0, i, 0)),
            pl.BlockSpec((2, 400, 16), lambda i: (0, i, 0)),
            pl.BlockSpec((1, 400, C), lambda i: (KC1 - 1, i, 0)),
            pl.BlockSpec((1, D_H1), lambda i: (0, 0)),
            pl.BlockSpec((D_H1, D_H2), lambda i: (0, 0)),
        ],
        out_specs=pl.BlockSpec((KC2, 400, C), lambda i: (0, i, 0)),
        out_shape=jax.ShapeDtypeStruct((KC2, N, C), jnp.float32),
    )(acc1, deg2, g1, b1, w2)


def _final_body(a_ref, deg_ref, b2_ref, o_ref):
    dinv = _dinv_of(deg_ref)
    acc = jnp.concatenate([a_ref[k] for k in range(KC2)], axis=1)
    o_ref[...] = acc * dinv + b2_ref[...]


def _final(acc2, deg2, b2):
    return pl.pallas_call(
        _final_body,
        grid=(25,),
        in_specs=[
            pl.BlockSpec((KC2, 400, C), lambda i: (0, i, 0)),
            pl.BlockSpec((2, 400, 16), lambda i: (0, i, 0)),
            pl.BlockSpec((1, D_H2), lambda i: (0, 0)),
        ],
        out_specs=pl.BlockSpec((400, D_H2), lambda i: (i, 0)),
        out_shape=jax.ShapeDtypeStruct((N, D_H2), jnp.float32),
    )(acc2, deg2, b2)


def kernel(x, edge_index, W1, b1, W2, b2):
    src = edge_index[0].astype(jnp.int32)
    dst = edge_index[1].astype(jnp.int32)
    npad = E_PAD - E
    src_p = jnp.concatenate([src, jnp.zeros((npad,), jnp.int32)])
    dst_p = jnp.concatenate([dst, jnp.full((npad,), TRASH, jnp.int32)])
    src16 = src_p.reshape(16, NB16, EB)
    dst16 = dst_p.reshape(16, NB16, EB)
    dst32 = dst_p.reshape(32, NB32, EB)

    deg2 = _deg_kernel(dst32)[:, :N, :]
    g1 = _mm1(x, W1, deg2)
    acc1 = _scatter_l1(g1, src16, dst16)
    g2 = _mm2(acc1, deg2, g1, b1.reshape(1, D_H1), W2)
    acc2 = _scatter_l2(g2, src16, dst16)
    return _final(acc2, deg2, b2.reshape(1, D_H2))


# speedup vs baseline: 1.3944x; 1.2379x over previous
import functools

import jax
import jax.numpy as jnp
from jax import lax
from jax.experimental import pallas as pl
from jax.experimental.pallas import tpu as pltpu
from jax.experimental.pallas import tpu_sc as plsc

N = 10000
E = 160000
D_IN = 128
D_H1 = 400
D_H2 = 800

C = 80
KC1 = D_H1 // C
KC2 = D_H2 // C

EB = 128
E_PAD = 163840
NB16 = E_PAD // 16 // EB
NB32 = E_PAD // 32 // EB
TRASH = N

ROWS_PER_TILE = N // 16

DEG_TILE = 626
DEG_ROWS = 16 * DEG_TILE

NBUF = 2
NQ = 4
HQ = 2
QB = NB16 // NQ
QGRP = QB // NBUF

_MESH = plsc.VectorSubcoreMesh(core_axis_name="c", subcore_axis_name="s")
_SC_PARAMS = pltpu.CompilerParams(use_tc_tiling_on_sc=False)


@functools.partial(
    pl.kernel,
    mesh=_MESH,
    out_type=jax.ShapeDtypeStruct((2, DEG_ROWS, 16), jnp.float32),
    scratch_types=[
        pltpu.VMEM_SHARED((DEG_ROWS, 16), jnp.float32),
        pltpu.VMEM((NB32, EB), jnp.int32),
        pltpu.VMEM((EB, 16), jnp.float32),
        pltpu.VMEM((DEG_TILE, 16), jnp.float32),
    ],
    compiler_params=_SC_PARAMS,
)
def _deg_kernel(dst_hbm, out_hbm, deg_sp, idx_v, ones_v, buf_v):
    c = lax.axis_index("c")
    s = lax.axis_index("s")
    w = c * 16 + s

    def fill_ones(i, _):
        ones_v[i, :] = jnp.full((16,), 1.0, jnp.float32)
        return 0

    lax.fori_loop(0, EB, fill_ones, 0)

    def fill_zero(i, _):
        buf_v[i, :] = jnp.zeros((16,), jnp.float32)
        return 0

    lax.fori_loop(0, DEG_TILE, fill_zero, 0)

    r0 = s * DEG_TILE
    pltpu.sync_copy(buf_v, deg_sp.at[pl.ds(r0, DEG_TILE)])
    pltpu.sync_copy(dst_hbm.at[w], idx_v)
    plsc.subcore_barrier()

    def edge(b, _):
        pltpu.sync_copy(ones_v, deg_sp.at[idx_v.at[b]], add=True)
        return 0

    lax.fori_loop(0, NB32, edge, 0)
    plsc.subcore_barrier()

    pltpu.sync_copy(deg_sp.at[pl.ds(r0, DEG_TILE)], buf_v)
    pltpu.sync_copy(buf_v, out_hbm.at[c].at[pl.ds(r0, DEG_TILE)])


def _make_scatter(kc_total):
    split_last = kc_total % 2 == 1
    kcc = kc_total // 2
    n_out = kc_total + 1 if split_last else kc_total

    @functools.partial(
        pl.kernel,
        mesh=_MESH,
        out_type=jax.ShapeDtypeStruct((n_out, N, C), jnp.float32),
        scratch_types=[
            pltpu.VMEM_SHARED((N, C), jnp.float32),
            pltpu.VMEM_SHARED((N + 1, C), jnp.float32),
            pltpu.VMEM((QB, EB), jnp.int32),
            pltpu.VMEM((QB, EB), jnp.int32),
            pltpu.VMEM((NBUF, EB, C), jnp.float32),
            pltpu.SemaphoreType.DMA((NBUF,)),
            pltpu.SemaphoreType.DMA((NBUF,)),
        ],
        compiler_params=_SC_PARAMS,
    )
    def scatter(g_hbm, src_hbm, dst_hbm, out_hbm,
                tab_sp, acc_sp, src_v, dst_v, row_v, sem_g, sem_s):
        c = lax.axis_index("c")
        s = lax.axis_index("s")
        r0 = s * ROWS_PER_TILE
        rows = pl.ds(r0, ROWS_PER_TILE)

        def run_chunk(k, out_slot, q_base, nq_local, hq_local):
            pltpu.sync_copy(g_hbm.at[k].at[rows], tab_sp.at[rows])
            pltpu.sync_copy(g_hbm.at[k].at[rows], acc_sp.at[rows])
            plsc.subcore_barrier()

            def g_start(i, b, from_hbm):
                src = (g_hbm.at[k] if from_hbm else tab_sp)
                pltpu.async_copy(src.at[src_v.at[b]], row_v.at[i],
                                 sem_g.at[i])

            def g_wait(i, b, from_hbm):
                src = (g_hbm.at[k] if from_hbm else tab_sp)
                pltpu.make_async_copy(src.at[src_v.at[b]], row_v.at[i],
                                      sem_g.at[i]).wait()

            def s_start(i, b):
                pltpu.async_copy(row_v.at[i], acc_sp.at[dst_v.at[b]],
                                 sem_s.at[i], add=True)

            def s_wait(i, b):
                pltpu.make_async_copy(row_v.at[i], acc_sp.at[dst_v.at[b]],
                                      sem_s.at[i]).wait()

            for q in range(nq_local):
                hbm_src = q < hq_local
                q_off = (q_base + q) * QB
                pltpu.sync_copy(src_hbm.at[s].at[pl.ds(q_off, QB)], src_v)
                pltpu.sync_copy(dst_hbm.at[s].at[pl.ds(q_off, QB)], dst_v)
                for i in range(NBUF):
                    g_start(i, i, hbm_src)

                def edge_grp(g, _, hbm_src=hbm_src):
                    for i in range(NBUF):
                        g_wait(i, g * NBUF + i, hbm_src)
                        s_start(i, g * NBUF + i)
                    for i in range(NBUF):
                        s_wait(i, g * NBUF + i)
                        g_start(i, (g + 1) * NBUF + i, hbm_src)
                    return 0

                lax.fori_loop(0, QGRP - 1, edge_grp, 0)
                for i in range(NBUF):
                    b = (QGRP - 1) * NBUF + i
                    g_wait(i, b, hbm_src)
                    s_start(i, b)
                for i in range(NBUF):
                    s_wait(i, (QGRP - 1) * NBUF + i)
            plsc.subcore_barrier()

            pltpu.sync_copy(acc_sp.at[rows], out_hbm.at[out_slot].at[rows])
            plsc.subcore_barrier()

        def chunk(kc, _):
            k = kc * 2 + c
            run_chunk(k, k, jnp.int32(0), NQ, HQ)
            return 0

        lax.fori_loop(0, kcc, chunk, 0)
        if split_last:
            run_chunk(kc_total - 1, kc_total - 1 + c, c * (NQ // 2),
                      NQ // 2, HQ // 2)

    return scatter


_scatter_l1 = _make_scatter(KC1)
_scatter_l2 = _make_scatter(KC2)


def _dinv_of(deg_ref):
    return lax.rsqrt(deg_ref[0, :, :1] + deg_ref[1, :, :1] + 1.0)


def _mm1_body(x_ref, w_ref, deg_ref, o_ref):
    dinv = _dinv_of(deg_ref)
    g = jnp.dot(x_ref[...], w_ref[...],
                preferred_element_type=jnp.float32) * dinv
    for k in range(KC1):
        o_ref[k] = g[:, k * C:(k + 1) * C]


def _mm1(x, w1, deg2):
    return pl.pallas_call(
        _mm1_body,
        grid=(25,),
        in_specs=[
            pl.BlockSpec((400, D_IN), lambda i: (i, 0)),
            pl.BlockSpec((D_IN, D_H1), lambda i: (0, 0)),
            pl.BlockSpec((2, 400, 16), lambda i: (0, i, 0)),
        ],
        out_specs=pl.BlockSpec((KC1, 400, C), lambda i: (0, i, 0)),
        out_shape=jax.ShapeDtypeStruct((KC1, N, C), jnp.float32),
    )(x, w1, deg2)


def _mm2_body(a_ref, deg_ref, g4_ref, b1_ref, w_ref, o_ref):
    dinv = _dinv_of(deg_ref)
    parts = [a_ref[k] for k in range(KC1 - 1)]
    parts.append(a_ref[KC1 - 1] + a_ref[KC1] - g4_ref[0])
    h = jnp.concatenate(parts, axis=1)
    h = jnp.maximum(h * dinv + b1_ref[...], 0.0)
    g = jnp.dot(h, w_ref[...], preferred_element_type=jnp.float32) * dinv
    for k in range(KC2):
        o_ref[k] = g[:, k * C:(k + 1) * C]


def _mm2(acc1, deg2, g1, b1, w2):
    return pl.pallas_call(
        _mm2_body,
        grid=(25,),
        in_specs=[
            pl.BlockSpec((KC1 + 1, 400, C), lambda i: (0, i, 0)),
            pl.BlockSpec((2, 400, 16), lambda i: (0, i, 0)),
            pl.BlockSpec((1, 400, C), lambda i: (KC1 - 1, i, 0)),
            pl.BlockSpec((1, D_H1), lambda i: (0, 0)),
            pl.BlockSpec((D_H1, D_H2), lambda i: (0, 0)),
        ],
        out_specs=pl.BlockSpec((KC2, 400, C), lambda i: (0, i, 0)),
        out_shape=jax.ShapeDtypeStruct((KC2, N, C), jnp.float32),
    )(acc1, deg2, g1, b1, w2)


def _final_body(a_ref, deg_ref, b2_ref, o_ref):
    dinv = _dinv_of(deg_ref)
    acc = jnp.concatenate([a_ref[k] for k in range(KC2)], axis=1)
    o_ref[...] = acc * dinv + b2_ref[...]


def _final(acc2, deg2, b2):
    return pl.pallas_call(
        _final_body,
        grid=(25,),
        in_specs=[
            pl.BlockSpec((KC2, 400, C), lambda i: (0, i, 0)),
            pl.BlockSpec((2, 400, 16), lambda i: (0, i, 0)),
            pl.BlockSpec((1, D_H2), lambda i: (0, 0)),
        ],
        out_specs=pl.BlockSpec((400, D_H2), lambda i: (i, 0)),
        out_shape=jax.ShapeDtypeStruct((N, D_H2), jnp.float32),
    )(acc2, deg2, b2)


def kernel(x, edge_index, W1, b1, W2, b2):
    src = edge_index[0].astype(jnp.int32)
    dst = edge_index[1].astype(jnp.int32)
    npad = E_PAD - E
    src_p = jnp.concatenate([src, jnp.zeros((npad,), jnp.int32)])
    dst_p = jnp.concatenate([dst, jnp.full((npad,), TRASH, jnp.int32)])
    src16 = src_p.reshape(16, NB16, EB)
    dst16 = dst_p.reshape(16, NB16, EB)
    dst32 = dst_p.reshape(32, NB32, EB)

    deg2 = _deg_kernel(dst32)[:, :N, :]
    g1 = _mm1(x, W1, deg2)
    acc1 = _scatter_l1(g1, src16, dst16)
    g2 = _mm2(acc1, deg2, g1, b1.reshape(1, D_H1), W2)
    acc2 = _scatter_l2(g2, src16, dst16)
    return _final(acc2, deg2, b2.reshape(1, D_H2))
